# baseline (speedup 1.0000x reference)
import functools

import jax
import jax.numpy as jnp
from jax import lax
from jax.experimental import pallas as pl
from jax.experimental.pallas import tpu as pltpu
from jax.experimental.pallas import tpu_sc as plsc

N = 50000
C = 128
E = 512000
NS = 16
K = 64
NCH = (E // NS) // K
NR = 4
RW = 12800
NP = NR * RW
TR = 128
RA = RW + TR
APS = RA // NS
RPS = RW // NS
FC = 32
BN = 1000
GRID = N // BN

F32 = jnp.float32
I32 = jnp.int32


def _mm_pre_body(x_ref, wrel_ref, wroot_ref, y_ref, z_ref):
    x = x_ref[...]
    y_ref[...] = jnp.dot(x, wrel_ref[...], preferred_element_type=F32)
    z_ref[...] = jnp.dot(x, wroot_ref[...], preferred_element_type=F32)


def _mm_pre(x, wrelT, wrootT):
    return pl.pallas_call(
        _mm_pre_body,
        grid=(GRID,),
        in_specs=[pl.BlockSpec((BN, C), lambda i: (i, 0)),
                  pl.BlockSpec((C, C), lambda i: (0, 0)),
                  pl.BlockSpec((C, C), lambda i: (0, 0))],
        out_specs=[pl.BlockSpec((BN, C), lambda i: (i, 0))] * 2,
        out_shape=[jax.ShapeDtypeStruct((N, C), F32)] * 2,
    )(x, wrelT, wrootT)


def _ln_relu(y, g, b):
    mu = jnp.mean(y, axis=1, keepdims=True)
    d = y - mu
    var = jnp.mean(d * d, axis=1, keepdims=True)
    return jnp.maximum(d * lax.rsqrt(var + 1e-5) * g + b, 0.0)


def _post_mm_body(s_ref, z_ref, b_ref, g_ref, bb_ref, wrel_ref, wroot_ref,
                  y_ref, z2_ref):
    y = s_ref[...] + b_ref[...] + z_ref[...]
    h = _ln_relu(y, g_ref[...], bb_ref[...])
    y_ref[...] = jnp.dot(h, wrel_ref[...], preferred_element_type=F32)
    z2_ref[...] = jnp.dot(h, wroot_ref[...], preferred_element_type=F32)


def _post_mm(s, z, b, g, bb, wrelT, wrootT):
    return pl.pallas_call(
        _post_mm_body,
        grid=(GRID,),
        in_specs=[pl.BlockSpec((BN, C), lambda i: (i, 0)),
                  pl.BlockSpec((BN, C), lambda i: (i, 0)),
                  pl.BlockSpec((1, C), lambda i: (0, 0)),
                  pl.BlockSpec((1, C), lambda i: (0, 0)),
                  pl.BlockSpec((1, C), lambda i: (0, 0)),
                  pl.BlockSpec((C, C), lambda i: (0, 0)),
                  pl.BlockSpec((C, C), lambda i: (0, 0))],
        out_specs=[pl.BlockSpec((BN, C), lambda i: (i, 0))] * 2,
        out_shape=[jax.ShapeDtypeStruct((N, C), F32)] * 2,
    )(s, z, b, g, bb, wrelT, wrootT)


def _post_body(s_ref, z_ref, b_ref, g_ref, bb_ref, out_ref):
    y = s_ref[...] + b_ref[...] + z_ref[...]
    out_ref[...] = _ln_relu(y, g_ref[...], bb_ref[...])


def _post(s, z, b, g, bb):
    return pl.pallas_call(
        _post_body,
        grid=(GRID,),
        in_specs=[pl.BlockSpec((BN, C), lambda i: (i, 0)),
                  pl.BlockSpec((BN, C), lambda i: (i, 0)),
                  pl.BlockSpec((1, C), lambda i: (0, 0)),
                  pl.BlockSpec((1, C), lambda i: (0, 0)),
                  pl.BlockSpec((1, C), lambda i: (0, 0))],
        out_specs=pl.BlockSpec((BN, C), lambda i: (i, 0)),
        out_shape=jax.ShapeDtypeStruct((N, C), F32),
    )(s, z, b, g, bb)


def _make_sc(with_counts):
    n_in = 3 if with_counts else 4
    out_type = [jax.ShapeDtypeStruct((NP, C), F32)]
    if with_counts:
        out_type += [jax.ShapeDtypeStruct((NP,), F32)]
    n_out = len(out_type)

    scratch = [
        pltpu.VMEM((K,), I32),
        pltpu.VMEM((K,), I32),
        pltpu.VMEM((K,), I32),
        pltpu.VMEM((K,), I32),
        pltpu.VMEM((K, C), F32),
        pltpu.VMEM((K, C), F32),
        pltpu.VMEM((FC, C), F32),
        pltpu.VMEM((16, C), F32),
        pltpu.VMEM((K,), I32),
        pltpu.VMEM((K,), I32),
        pltpu.VMEM((K,), F32),
        pltpu.VMEM((RPS,), F32),
        pltpu.VMEM((APS + 8,), F32),
        pltpu.VMEM_SHARED((RA, C), F32),
        pltpu.VMEM_SHARED((RA,), F32),
        pltpu.SemaphoreType.DMA,
        pltpu.SemaphoreType.DMA,
        pltpu.SemaphoreType.DMA,
        pltpu.SemaphoreType.DMA,
    ]

    mesh = plsc.VectorSubcoreMesh(core_axis_name="c", subcore_axis_name="s")

    @functools.partial(pl.kernel, out_type=out_type, mesh=mesh,
                       scratch_types=scratch)
    def sc_agg(*refs):
        ins = refs[:n_in]
        yf, srch, dsth = ins[:3]
        outs = refs[n_in:n_in + n_out]
        sm = outs[0]
        (sbuf0, sbuf1, dbuf0, dbuf1, rb0, rb1, stage, zbuf, dl0, dl1, onesb,
         cbuf, z1, acc, cnt_sp, semg0, semg1, semid0, semid1) = \
            refs[n_in + n_out:]

        c = lax.axis_index("c")
        s = lax.axis_index("s")
        e0 = s * (E // NS)

        z16 = jnp.zeros((16,), F32)
        for v in range(K // 16):
            onesb[pl.ds(16 * v, 16)] = jnp.ones((16,), F32)
        for v in range((APS + 8) // 16):
            z1[pl.ds(16 * v, 16)] = z16
        for r in range(16):
            for v in range(C // 16):
                zbuf[r, pl.ds(16 * v, 16)] = z16
        for r in range(16):
            for v in range(C // 16):
                zbuf[r, pl.ds(16 * v, 16)] = z16
        iota16 = lax.iota(I32, 16)
        trash = RW + lax.rem(16 * s + iota16, jnp.int32(TR))

        def dst_local(dbuf, base, dl):
            for v in range(K // 16):
                dv = dbuf[pl.ds(16 * v, 16)]
                t = dv - base
                m = (t >= 0) & (t < RW)
                dl[pl.ds(16 * v, 16)] = jnp.where(m, t, trash)

        def run_rel(cnt_out, recip_in):
            def idload(i, sb, db, sem):
                ic = lax.rem(i, jnp.int32(NCH))
                pltpu.async_copy(srch.at[pl.ds(e0 + ic * K, K)], sb, sem)
                pltpu.async_copy(dsth.at[pl.ds(e0 + ic * K, K)], db, sem)

            def idwait(sb, db, sem):
                pltpu.make_async_copy(srch.at[pl.ds(e0, K)], sb, sem).wait()
                pltpu.make_async_copy(dsth.at[pl.ds(e0, K)], db, sem).wait()

            def gwait(rb, sem):
                pltpu.make_async_copy(yf.at[sbuf0], rb, sem).wait()

            for j in range(2):
                b = 2 * c + j
                base = b * RW
                def zbody(k2, carry):
                    pltpu.sync_copy(zbuf, acc.at[pl.ds(s * APS + k2 * 16, 16)])
                    return carry

                lax.fori_loop(0, APS // 16, zbody, 0)
                pltpu.sync_copy(zbuf.at[pl.ds(0, 8)],
                                acc.at[pl.ds(s * APS + (APS // 16) * 16, 8)])
                if with_counts:
                    pltpu.sync_copy(z1.at[pl.ds(0, APS)],
                                    cnt_sp.at[pl.ds(s * APS, APS)])
                plsc.subcore_barrier()

                idload(0, sbuf0, dbuf0, semid0)
                idload(1, sbuf1, dbuf1, semid1)
                idwait(sbuf0, dbuf0, semid0)
                pltpu.async_copy(yf.at[sbuf0], rb0, semg0)

                def body(t, carry):
                    i0 = 2 * t
                    idwait(sbuf1, dbuf1, semid1)
                    pltpu.async_copy(yf.at[sbuf1], rb1, semg1)
                    dst_local(dbuf0, base, dl0)
                    gwait(rb0, semg0)
                    idload(i0 + 2, sbuf0, dbuf0, semid0)
                    pltpu.sync_copy(rb0, acc.at[dl0], add=True)
                    if with_counts:
                        pltpu.sync_copy(onesb, cnt_sp.at[dl0], add=True)
                    dst_local(dbuf1, base, dl1)
                    gwait(rb1, semg1)
                    idload(i0 + 3, sbuf1, dbuf1, semid1)
                    pltpu.sync_copy(rb1, acc.at[dl1], add=True)
                    if with_counts:
                        pltpu.sync_copy(onesb, cnt_sp.at[dl1], add=True)
                    idwait(sbuf0, dbuf0, semid0)
                    pltpu.async_copy(yf.at[sbuf0], rb0, semg0)
                    return carry

                lax.fori_loop(0, NCH // 2, body, 0)
                gwait(rb0, semg0)
                idwait(sbuf1, dbuf1, semid1)
                plsc.subcore_barrier()

                if with_counts:
                    pltpu.sync_copy(cnt_sp.at[pl.ds(s * RPS, RPS)], cbuf)

                    def rbody(g, carry):
                        cv = cbuf[pl.ds(16 * g, 16)]
                        cbuf[pl.ds(16 * g, 16)] = 1.0 / jnp.maximum(cv, 1.0)
                        return carry

                    lax.fori_loop(0, RPS // 16, rbody, 0)
                    pltpu.sync_copy(cbuf, cnt_out.at[pl.ds(base + s * RPS, RPS)])
                else:
                    pltpu.sync_copy(recip_in.at[pl.ds(base + s * RPS, RPS)], cbuf)

                def fbody(k2, carry):
                    pltpu.sync_copy(acc.at[pl.ds(s * RPS + k2 * FC, FC)], stage)

                    def mbody(g, carry2):
                        rv = cbuf[pl.ds(k2 * FC + 16 * g, 16)]
                        for jr in range(16):
                            rs = jnp.broadcast_to(rv[jr], (16,))
                            row = 16 * g + jr
                            for v in range(C // 16):
                                sl = pl.ds(16 * v, 16)
                                stage[row, sl] = stage[row, sl] * rs
                        return carry2

                    lax.fori_loop(0, FC // 16, mbody, 0)
                    pltpu.sync_copy(
                        stage, sm.at[pl.ds(base + s * RPS + k2 * FC, FC)])
                    return carry

                lax.fori_loop(0, RPS // FC, fbody, 0)
                plsc.subcore_barrier()

        def zbody(k2, carry):
            pltpu.sync_copy(zbuf, acc.at[pl.ds(s * RPS + k2 * 16, 16)])
            return carry

        lax.fori_loop(0, RPS // 16, zbody, 0)

        if with_counts:
            run_rel(outs[1], None)
        else:
            run_rel(None, ins[3])

    return sc_agg


_sc_agg_l0 = _make_sc(True)
_sc_agg_l1 = _make_sc(False)


def kernel(x_user, x_item, ei_user_item, ei_item_user,
           W_rel_0_ui, b_rel_0_ui, W_root_0_ui,
           W_rel_0_iu, b_rel_0_iu, W_root_0_iu,
           ln_g_0_user, ln_b_0_user, ln_g_0_item, ln_b_0_item,
           W_rel_1_ui, b_rel_1_ui, W_root_1_ui,
           W_rel_1_iu, b_rel_1_iu, W_root_1_iu,
           ln_g_1_user, ln_b_1_user, ln_g_1_item, ln_b_1_item):
    src_ui, dst_ui = ei_user_item[0], ei_user_item[1]
    src_iu, dst_iu = ei_item_user[0], ei_item_user[1]
    r2 = lambda v: v.reshape(1, C)

    yu, zu = _mm_pre(x_user, W_rel_0_ui.T, W_root_0_iu.T)
    yi, zi = _mm_pre(x_item, W_rel_0_iu.T, W_root_0_ui.T)

    smi, recip_i = _sc_agg_l0(yu, src_ui, dst_ui)
    smu, recip_u = _sc_agg_l0(yi, src_iu, dst_iu)
    yi1, zi1 = _post_mm(smi, zi, r2(b_rel_0_ui),
                        r2(ln_g_0_item), r2(ln_b_0_item),
                        W_rel_1_iu.T, W_root_1_ui.T)
    smu1 = _sc_agg_l1(yi1, src_iu, dst_iu, recip_u)[0]
    yu1, zu1 = _post_mm(smu, zu, r2(b_rel_0_iu),
                        r2(ln_g_0_user), r2(ln_b_0_user),
                        W_rel_1_ui.T, W_root_1_iu.T)
    smi1 = _sc_agg_l1(yu1, src_ui, dst_ui, recip_i)[0]
    out_user = _post(smu1, zu1, r2(b_rel_1_iu),
                     r2(ln_g_1_user), r2(ln_b_1_user))
    out_item = _post(smi1, zi1, r2(b_rel_1_ui),
                     r2(ln_g_1_item), r2(ln_b_1_item))
    return (out_user, out_item)


# speedup vs baseline: 1.0045x; 1.0045x over previous
import functools

import jax
import jax.numpy as jnp
from jax import lax
from jax.experimental import pallas as pl
from jax.experimental.pallas import tpu as pltpu
from jax.experimental.pallas import tpu_sc as plsc

N = 50000
C = 128
E = 512000
NS = 16
K = 64
NCH = (E // NS) // K
NR = 4
RW = 12800
NP = NR * RW
TR = 128
RA = RW + TR
APS = RA // NS
RPS = RW // NS
FC = 32
BN = 1000
GRID = N // BN

F32 = jnp.float32
I32 = jnp.int32


def _mm_pre_body(x_ref, wrel_ref, wroot_ref, y_ref, z_ref):
    x = x_ref[...]
    y_ref[...] = jnp.dot(x, wrel_ref[...], preferred_element_type=F32)
    z_ref[...] = jnp.dot(x, wroot_ref[...], preferred_element_type=F32)


def _mm_pre(x, wrelT, wrootT):
    return pl.pallas_call(
        _mm_pre_body,
        grid=(GRID,),
        in_specs=[pl.BlockSpec((BN, C), lambda i: (i, 0)),
                  pl.BlockSpec((C, C), lambda i: (0, 0)),
                  pl.BlockSpec((C, C), lambda i: (0, 0))],
        out_specs=[pl.BlockSpec((BN, C), lambda i: (i, 0))] * 2,
        out_shape=[jax.ShapeDtypeStruct((N, C), F32)] * 2,
    )(x, wrelT, wrootT)


def _ln_relu(y, g, b):
    mu = jnp.mean(y, axis=1, keepdims=True)
    d = y - mu
    var = jnp.mean(d * d, axis=1, keepdims=True)
    return jnp.maximum(d * lax.rsqrt(var + 1e-5) * g + b, 0.0)


def _post_mm_body(s_ref, z_ref, b_ref, g_ref, bb_ref, wrel_ref, wroot_ref,
                  y_ref, z2_ref):
    y = s_ref[...] + b_ref[...] + z_ref[...]
    h = _ln_relu(y, g_ref[...], bb_ref[...])
    y_ref[...] = jnp.dot(h, wrel_ref[...], preferred_element_type=F32)
    z2_ref[...] = jnp.dot(h, wroot_ref[...], preferred_element_type=F32)


def _post_mm(s, z, b, g, bb, wrelT, wrootT):
    return pl.pallas_call(
        _post_mm_body,
        grid=(GRID,),
        in_specs=[pl.BlockSpec((BN, C), lambda i: (i, 0)),
                  pl.BlockSpec((BN, C), lambda i: (i, 0)),
                  pl.BlockSpec((1, C), lambda i: (0, 0)),
                  pl.BlockSpec((1, C), lambda i: (0, 0)),
                  pl.BlockSpec((1, C), lambda i: (0, 0)),
                  pl.BlockSpec((C, C), lambda i: (0, 0)),
                  pl.BlockSpec((C, C), lambda i: (0, 0))],
        out_specs=[pl.BlockSpec((BN, C), lambda i: (i, 0))] * 2,
        out_shape=[jax.ShapeDtypeStruct((N, C), F32)] * 2,
    )(s, z, b, g, bb, wrelT, wrootT)


def _post_body(s_ref, z_ref, b_ref, g_ref, bb_ref, out_ref):
    y = s_ref[...] + b_ref[...] + z_ref[...]
    out_ref[...] = _ln_relu(y, g_ref[...], bb_ref[...])


def _post(s, z, b, g, bb):
    return pl.pallas_call(
        _post_body,
        grid=(GRID,),
        in_specs=[pl.BlockSpec((BN, C), lambda i: (i, 0)),
                  pl.BlockSpec((BN, C), lambda i: (i, 0)),
                  pl.BlockSpec((1, C), lambda i: (0, 0)),
                  pl.BlockSpec((1, C), lambda i: (0, 0)),
                  pl.BlockSpec((1, C), lambda i: (0, 0))],
        out_specs=pl.BlockSpec((BN, C), lambda i: (i, 0)),
        out_shape=jax.ShapeDtypeStruct((N, C), F32),
    )(s, z, b, g, bb)


def _make_sc(with_counts):
    n_in = 3 if with_counts else 4
    out_type = [jax.ShapeDtypeStruct((NP, C), F32)]
    if with_counts:
        out_type += [jax.ShapeDtypeStruct((NP,), F32)]
    n_out = len(out_type)

    scratch = [
        pltpu.VMEM((K,), I32),
        pltpu.VMEM((K,), I32),
        pltpu.VMEM((K,), I32),
        pltpu.VMEM((K,), I32),
        pltpu.VMEM((K, C), F32),
        pltpu.VMEM((K, C), F32),
        pltpu.VMEM((FC, C), F32),
        pltpu.VMEM((16, C), F32),
        pltpu.VMEM((K,), I32),
        pltpu.VMEM((K,), I32),
        pltpu.VMEM((K,), F32),
        pltpu.VMEM((RPS,), F32),
        pltpu.VMEM((APS + 8,), F32),
        pltpu.VMEM_SHARED((RA, C), F32),
        pltpu.VMEM_SHARED((RA,), F32),
        pltpu.SemaphoreType.DMA,
        pltpu.SemaphoreType.DMA,
        pltpu.SemaphoreType.DMA,
        pltpu.SemaphoreType.DMA,
    ]

    mesh = plsc.VectorSubcoreMesh(core_axis_name="c", subcore_axis_name="s")

    @functools.partial(pl.kernel, out_type=out_type, mesh=mesh,
                       scratch_types=scratch)
    def sc_agg(*refs):
        ins = refs[:n_in]
        yf, srch, dsth = ins[:3]
        outs = refs[n_in:n_in + n_out]
        sm = outs[0]
        (sbuf0, sbuf1, dbuf0, dbuf1, rb0, rb1, stage, zbuf, dl0, dl1, onesb,
         cbuf, z1, acc, cnt_sp, semg0, semg1, semid0, semid1) = \
            refs[n_in + n_out:]

        c = lax.axis_index("c")
        s = lax.axis_index("s")
        e0 = s * (E // NS)

        z16 = jnp.zeros((16,), F32)
        for v in range(K // 16):
            onesb[pl.ds(16 * v, 16)] = jnp.ones((16,), F32)
        for v in range((APS + 8) // 16):
            z1[pl.ds(16 * v, 16)] = z16
        for r in range(16):
            for v in range(C // 16):
                zbuf[r, pl.ds(16 * v, 16)] = z16
        for r in range(16):
            for v in range(C // 16):
                zbuf[r, pl.ds(16 * v, 16)] = z16
        iota16 = lax.iota(I32, 16)
        trash = RW + lax.rem(16 * s + iota16, jnp.int32(TR))

        def dst_local(dbuf, base, dl):
            for v in range(K // 16):
                dv = dbuf[pl.ds(16 * v, 16)]
                t = dv - base
                m = (t >= 0) & (t < RW)
                dl[pl.ds(16 * v, 16)] = jnp.where(m, t, trash)

        def run_rel(cnt_out, recip_in):
            def idload(i, sb, db, sem):
                ic = lax.rem(i, jnp.int32(NCH))
                pltpu.async_copy(srch.at[pl.ds(e0 + ic * K, K)], sb, sem)
                pltpu.async_copy(dsth.at[pl.ds(e0 + ic * K, K)], db, sem)

            def idwait(sb, db, sem):
                pltpu.make_async_copy(srch.at[pl.ds(e0, K)], sb, sem).wait()
                pltpu.make_async_copy(dsth.at[pl.ds(e0, K)], db, sem).wait()

            def gwait(rb, sem):
                pltpu.make_async_copy(yf.at[sbuf0], rb, sem).wait()

            for j in range(2):
                b = 2 * c + j
                base = b * RW
                def zbody(k2, carry):
                    pltpu.sync_copy(zbuf, acc.at[pl.ds(s * APS + k2 * 16, 16)])
                    return carry

                lax.fori_loop(0, APS // 16, zbody, 0)
                pltpu.sync_copy(zbuf.at[pl.ds(0, 8)],
                                acc.at[pl.ds(s * APS + (APS // 16) * 16, 8)])
                if with_counts:
                    pltpu.sync_copy(z1.at[pl.ds(0, APS)],
                                    cnt_sp.at[pl.ds(s * APS, APS)])
                plsc.subcore_barrier()

                idload(0, sbuf0, dbuf0, semid0)
                idload(1, sbuf1, dbuf1, semid1)
                idwait(sbuf0, dbuf0, semid0)
                pltpu.async_copy(yf.at[sbuf0], rb0, semg0)

                def body(t, carry):
                    i0 = 2 * t
                    idwait(sbuf1, dbuf1, semid1)
                    pltpu.async_copy(yf.at[sbuf1], rb1, semg1)
                    dst_local(dbuf0, base, dl0)
                    gwait(rb0, semg0)
                    idload(i0 + 2, sbuf0, dbuf0, semid0)
                    pltpu.sync_copy(rb0, acc.at[dl0], add=True)
                    if with_counts:
                        pltpu.sync_copy(onesb, cnt_sp.at[dl0], add=True)
                    dst_local(dbuf1, base, dl1)
                    gwait(rb1, semg1)
                    idload(i0 + 3, sbuf1, dbuf1, semid1)
                    pltpu.sync_copy(rb1, acc.at[dl1], add=True)
                    if with_counts:
                        pltpu.sync_copy(onesb, cnt_sp.at[dl1], add=True)
                    idwait(sbuf0, dbuf0, semid0)
                    pltpu.async_copy(yf.at[sbuf0], rb0, semg0)
                    return carry

                lax.fori_loop(0, NCH // 2, body, 0)
                gwait(rb0, semg0)
                idwait(sbuf1, dbuf1, semid1)
                plsc.subcore_barrier()

                if with_counts:
                    pltpu.sync_copy(cnt_sp.at[pl.ds(s * RPS, RPS)], cbuf)

                    def rbody(g, carry):
                        cv = cbuf[pl.ds(16 * g, 16)]
                        cbuf[pl.ds(16 * g, 16)] = 1.0 / jnp.maximum(cv, 1.0)
                        return carry

                    lax.fori_loop(0, RPS // 16, rbody, 0)
                    pltpu.sync_copy(cbuf, cnt_out.at[pl.ds(base + s * RPS, RPS)])
                else:
                    pltpu.sync_copy(recip_in.at[pl.ds(base + s * RPS, RPS)], cbuf)

                def fbody(k2, carry):
                    pltpu.sync_copy(acc.at[pl.ds(s * RPS + k2 * FC, FC)], stage)

                    def mbody(g, carry2):
                        rv = cbuf[pl.ds(k2 * FC + 16 * g, 16)]
                        for jr in range(16):
                            rs = jnp.broadcast_to(rv[jr], (16,))
                            row = 16 * g + jr
                            for v in range(C // 16):
                                sl = pl.ds(16 * v, 16)
                                stage[row, sl] = stage[row, sl] * rs
                        return carry2

                    lax.fori_loop(0, FC // 16, mbody, 0)
                    pltpu.sync_copy(
                        stage, sm.at[pl.ds(base + s * RPS + k2 * FC, FC)])
                    return carry

                lax.fori_loop(0, RPS // FC, fbody, 0)
                plsc.subcore_barrier()

        if with_counts:
            run_rel(outs[1], None)
        else:
            run_rel(None, ins[3])

    return sc_agg


_sc_agg_l0 = _make_sc(True)
_sc_agg_l1 = _make_sc(False)


def kernel(x_user, x_item, ei_user_item, ei_item_user,
           W_rel_0_ui, b_rel_0_ui, W_root_0_ui,
           W_rel_0_iu, b_rel_0_iu, W_root_0_iu,
           ln_g_0_user, ln_b_0_user, ln_g_0_item, ln_b_0_item,
           W_rel_1_ui, b_rel_1_ui, W_root_1_ui,
           W_rel_1_iu, b_rel_1_iu, W_root_1_iu,
           ln_g_1_user, ln_b_1_user, ln_g_1_item, ln_b_1_item):
    src_ui, dst_ui = ei_user_item[0], ei_user_item[1]
    src_iu, dst_iu = ei_item_user[0], ei_item_user[1]
    r2 = lambda v: v.reshape(1, C)

    yu, zu = _mm_pre(x_user, W_rel_0_ui.T, W_root_0_iu.T)
    yi, zi = _mm_pre(x_item, W_rel_0_iu.T, W_root_0_ui.T)

    smi, recip_i = _sc_agg_l0(yu, src_ui, dst_ui)
    smu, recip_u = _sc_agg_l0(yi, src_iu, dst_iu)
    yi1, zi1 = _post_mm(smi, zi, r2(b_rel_0_ui),
                        r2(ln_g_0_item), r2(ln_b_0_item),
                        W_rel_1_iu.T, W_root_1_ui.T)
    smu1 = _sc_agg_l1(yi1, src_iu, dst_iu, recip_u)[0]
    yu1, zu1 = _post_mm(smu, zu, r2(b_rel_0_iu),
                        r2(ln_g_0_user), r2(ln_b_0_user),
                        W_rel_1_ui.T, W_root_1_iu.T)
    smi1 = _sc_agg_l1(yu1, src_ui, dst_ui, recip_i)[0]
    out_user = _post(smu1, zu1, r2(b_rel_1_iu),
                     r2(ln_g_1_user), r2(ln_b_1_user))
    out_item = _post(smi1, zi1, r2(b_rel_1_ui),
                     r2(ln_g_1_item), r2(ln_b_1_item))
    return (out_user, out_item)


# speedup vs baseline: 1.3500x; 1.3439x over previous
import functools

import jax
import jax.numpy as jnp
from jax import lax
from jax.experimental import pallas as pl
from jax.experimental.pallas import tpu as pltpu
from jax.experimental.pallas import tpu_sc as plsc

N = 50000
C = 128
E = 512000
NS = 16
K = 64
NCH = (E // NS) // K
NR = 4
RW = 12800
NP = NR * RW
TR = 128
RA = RW + TR
APS = RA // NS
RPS = RW // NS
FC = 32
BN = 1000
GRID = N // BN

F32 = jnp.float32
I32 = jnp.int32


def _mm_pre_body(x_ref, wrel_ref, wroot_ref, y_ref, z_ref):
    x = x_ref[...]
    y_ref[...] = jnp.dot(x, wrel_ref[...], preferred_element_type=F32)
    z_ref[...] = jnp.dot(x, wroot_ref[...], preferred_element_type=F32)


def _mm_pre(x, wrelT, wrootT):
    return pl.pallas_call(
        _mm_pre_body,
        grid=(GRID,),
        in_specs=[pl.BlockSpec((BN, C), lambda i: (i, 0)),
                  pl.BlockSpec((C, C), lambda i: (0, 0)),
                  pl.BlockSpec((C, C), lambda i: (0, 0))],
        out_specs=[pl.BlockSpec((BN, C), lambda i: (i, 0))] * 2,
        out_shape=[jax.ShapeDtypeStruct((N, C), F32)] * 2,
    )(x, wrelT, wrootT)


def _ln_relu(y, g, b):
    mu = jnp.mean(y, axis=1, keepdims=True)
    d = y - mu
    var = jnp.mean(d * d, axis=1, keepdims=True)
    return jnp.maximum(d * lax.rsqrt(var + 1e-5) * g + b, 0.0)


def _post_mm_body(s_ref, z_ref, b_ref, g_ref, bb_ref, wrel_ref, wroot_ref,
                  y_ref, z2_ref):
    y = s_ref[...] + b_ref[...] + z_ref[...]
    h = _ln_relu(y, g_ref[...], bb_ref[...])
    y_ref[...] = jnp.dot(h, wrel_ref[...], preferred_element_type=F32)
    z2_ref[...] = jnp.dot(h, wroot_ref[...], preferred_element_type=F32)


def _post_mm(s, z, b, g, bb, wrelT, wrootT):
    return pl.pallas_call(
        _post_mm_body,
        grid=(GRID,),
        in_specs=[pl.BlockSpec((BN, C), lambda i: (i, 0)),
                  pl.BlockSpec((BN, C), lambda i: (i, 0)),
                  pl.BlockSpec((1, C), lambda i: (0, 0)),
                  pl.BlockSpec((1, C), lambda i: (0, 0)),
                  pl.BlockSpec((1, C), lambda i: (0, 0)),
                  pl.BlockSpec((C, C), lambda i: (0, 0)),
                  pl.BlockSpec((C, C), lambda i: (0, 0))],
        out_specs=[pl.BlockSpec((BN, C), lambda i: (i, 0))] * 2,
        out_shape=[jax.ShapeDtypeStruct((N, C), F32)] * 2,
    )(s, z, b, g, bb, wrelT, wrootT)


def _post_body(s_ref, z_ref, b_ref, g_ref, bb_ref, out_ref):
    y = s_ref[...] + b_ref[...] + z_ref[...]
    out_ref[...] = _ln_relu(y, g_ref[...], bb_ref[...])


def _post(s, z, b, g, bb):
    return pl.pallas_call(
        _post_body,
        grid=(GRID,),
        in_specs=[pl.BlockSpec((BN, C), lambda i: (i, 0)),
                  pl.BlockSpec((BN, C), lambda i: (i, 0)),
                  pl.BlockSpec((1, C), lambda i: (0, 0)),
                  pl.BlockSpec((1, C), lambda i: (0, 0)),
                  pl.BlockSpec((1, C), lambda i: (0, 0))],
        out_specs=pl.BlockSpec((BN, C), lambda i: (i, 0)),
        out_shape=jax.ShapeDtypeStruct((N, C), F32),
    )(s, z, b, g, bb)


def _make_sc(with_counts):
    n_in = 3 if with_counts else 4
    out_type = [jax.ShapeDtypeStruct((NP, C), F32)]
    if with_counts:
        out_type += [jax.ShapeDtypeStruct((NP,), F32)]
    n_out = len(out_type)

    scratch = [
        pltpu.VMEM((5 * K,), I32),
        pltpu.VMEM((5 * K,), I32),
        pltpu.VMEM((5 * K,), I32),
        pltpu.VMEM((5 * K,), I32),
        pltpu.VMEM((K, C), F32),
        pltpu.VMEM((K, C), F32),
        pltpu.VMEM((FC, C), F32),
        pltpu.VMEM((16, C), F32),
        pltpu.VMEM((K,), I32),
        pltpu.VMEM((K,), I32),
        pltpu.VMEM((K,), F32),
        pltpu.VMEM((RPS,), F32),
        pltpu.VMEM((APS + 8,), F32),
        pltpu.VMEM_SHARED((RA, C), F32),
        pltpu.VMEM_SHARED((RA,), F32),
        pltpu.SemaphoreType.DMA,
        pltpu.SemaphoreType.DMA,
        pltpu.SemaphoreType.DMA,
        pltpu.SemaphoreType.DMA,
    ]

    mesh = plsc.VectorSubcoreMesh(core_axis_name="c", subcore_axis_name="s")

    @functools.partial(pl.kernel, out_type=out_type, mesh=mesh,
                       scratch_types=scratch)
    def sc_agg(*refs):
        ins = refs[:n_in]
        yf, srch, dsth = ins[:3]
        outs = refs[n_in:n_in + n_out]
        sm = outs[0]
        (sbuf0, sbuf1, dbuf0, dbuf1, rb0, rb1, stage, zbuf, dl0, dl1, onesb,
         cbuf, z1, acc, cnt_sp, semg0, semg1, semid0, semid1) = \
            refs[n_in + n_out:]

        c = lax.axis_index("c")
        s = lax.axis_index("s")
        e0 = s * (E // NS)

        z16 = jnp.zeros((16,), F32)
        for v in range(K // 16):
            onesb[pl.ds(16 * v, 16)] = jnp.ones((16,), F32)
        for v in range((APS + 8) // 16):
            z1[pl.ds(16 * v, 16)] = z16
        for r in range(16):
            for v in range(C // 16):
                zbuf[r, pl.ds(16 * v, 16)] = z16
        for r in range(16):
            for v in range(C // 16):
                zbuf[r, pl.ds(16 * v, 16)] = z16
        iota16 = lax.iota(I32, 16)
        trash = RW + lax.rem(16 * s + iota16, jnp.int32(TR))

        def dst_local(dbuf, base, dl):
            for v in range(K // 16):
                dv = dbuf[pl.ds(16 * v, 16)]
                t = dv - base
                m = (t >= 0) & (t < RW)
                dl[pl.ds(16 * v, 16)] = jnp.where(m, t, trash)

        def run_rel(cnt_out, recip_in):
            NIB = 5 * K
            NGRP = (E // NS) // NIB

            def idload(grp, sb, db, sem):
                ic = lax.rem(grp, jnp.int32(NGRP)) * NIB
                pltpu.async_copy(srch.at[pl.ds(e0 + ic, NIB)], sb, sem)
                pltpu.async_copy(dsth.at[pl.ds(e0 + ic, NIB)], db, sem)

            def idwait(sb, db, sem):
                pltpu.make_async_copy(srch.at[pl.ds(e0, NIB)], sb, sem).wait()
                pltpu.make_async_copy(dsth.at[pl.ds(e0, NIB)], db, sem).wait()

            def gissue(sb, qq, rb, sem):
                pltpu.async_copy(yf.at[sb.at[pl.ds(qq * K, K)]], rb, sem)

            def gwait(rb, sem):
                pltpu.make_async_copy(yf.at[sbuf0.at[pl.ds(0, K)]], rb,
                                      sem).wait()

            for j in range(2):
                b = 2 * c + j
                base = b * RW
                def zbody(k2, carry):
                    pltpu.sync_copy(zbuf, acc.at[pl.ds(s * APS + k2 * 16, 16)])
                    return carry

                lax.fori_loop(0, APS // 16, zbody, 0)
                pltpu.sync_copy(zbuf.at[pl.ds(0, 8)],
                                acc.at[pl.ds(s * APS + (APS // 16) * 16, 8)])
                if with_counts:
                    pltpu.sync_copy(z1.at[pl.ds(0, APS)],
                                    cnt_sp.at[pl.ds(s * APS, APS)])
                plsc.subcore_barrier()

                def chunk_dl(db, qq, dl):
                    for v in range(K // 16):
                        dv = db[pl.ds(qq * K + 16 * v, 16)]
                        t2 = dv - base
                        m = (t2 >= 0) & (t2 < RW)
                        dl[pl.ds(16 * v, 16)] = jnp.where(m, t2, trash)

                def scat(rb, dl):
                    pltpu.sync_copy(rb, acc.at[dl], add=True)
                    if with_counts:
                        pltpu.sync_copy(onesb, cnt_sp.at[dl], add=True)

                idload(0, sbuf0, dbuf0, semid0)
                idload(1, sbuf1, dbuf1, semid1)
                idwait(sbuf0, dbuf0, semid0)
                gissue(sbuf0, 0, rb0, semg0)

                def body(t, carry):
                    rbs = (rb0, rb1)
                    sgs = (semg0, semg1)
                    dls = (dl0, dl1)
                    for q in range(10):
                        nb = sbuf0 if q < 4 or q == 9 else sbuf1
                        nq = (q + 1) % 5
                        cb = dbuf0 if q < 5 else dbuf1
                        cq = q % 5
                        rw_, rn_ = rbs[q % 2], rbs[(q + 1) % 2]
                        sw_, sn_ = sgs[q % 2], sgs[(q + 1) % 2]
                        if q == 4:
                            idwait(sbuf1, dbuf1, semid1)
                        if q == 9:
                            idwait(sbuf0, dbuf0, semid0)
                        gissue(nb, nq, rn_, sn_)
                        chunk_dl(cb, cq, dls[q % 2])
                        gwait(rw_, sw_)
                        if q == 4:
                            idload(2 * t + 2, sbuf0, dbuf0, semid0)
                        if q == 9:
                            idload(2 * t + 3, sbuf1, dbuf1, semid1)
                        scat(rw_, dls[q % 2])
                    return carry

                lax.fori_loop(0, NCH // 10, body, 0)
                gwait(rb0, semg0)
                idwait(sbuf1, dbuf1, semid1)
                plsc.subcore_barrier()

                if with_counts:
                    pltpu.sync_copy(cnt_sp.at[pl.ds(s * RPS, RPS)], cbuf)

                    def rbody(g, carry):
                        cv = cbuf[pl.ds(16 * g, 16)]
                        cbuf[pl.ds(16 * g, 16)] = 1.0 / jnp.maximum(cv, 1.0)
                        return carry

                    lax.fori_loop(0, RPS // 16, rbody, 0)
                    pltpu.sync_copy(cbuf, cnt_out.at[pl.ds(base + s * RPS, RPS)])
                else:
                    pltpu.sync_copy(recip_in.at[pl.ds(base + s * RPS, RPS)], cbuf)

                def fbody(k2, carry):
                    pltpu.sync_copy(acc.at[pl.ds(s * RPS + k2 * FC, FC)], stage)

                    def mbody(g, carry2):
                        rv = cbuf[pl.ds(k2 * FC + 16 * g, 16)]
                        for jr in range(16):
                            rs = jnp.broadcast_to(rv[jr], (16,))
                            row = 16 * g + jr
                            for v in range(C // 16):
                                sl = pl.ds(16 * v, 16)
                                stage[row, sl] = stage[row, sl] * rs
                        return carry2

                    lax.fori_loop(0, FC // 16, mbody, 0)
                    pltpu.sync_copy(
                        stage, sm.at[pl.ds(base + s * RPS + k2 * FC, FC)])
                    return carry

                lax.fori_loop(0, RPS // FC, fbody, 0)
                plsc.subcore_barrier()

        if with_counts:
            run_rel(outs[1], None)
        else:
            run_rel(None, ins[3])

    return sc_agg


_sc_agg_l0 = _make_sc(True)
_sc_agg_l1 = _make_sc(False)


def kernel(x_user, x_item, ei_user_item, ei_item_user,
           W_rel_0_ui, b_rel_0_ui, W_root_0_ui,
           W_rel_0_iu, b_rel_0_iu, W_root_0_iu,
           ln_g_0_user, ln_b_0_user, ln_g_0_item, ln_b_0_item,
           W_rel_1_ui, b_rel_1_ui, W_root_1_ui,
           W_rel_1_iu, b_rel_1_iu, W_root_1_iu,
           ln_g_1_user, ln_b_1_user, ln_g_1_item, ln_b_1_item):
    src_ui, dst_ui = ei_user_item[0], ei_user_item[1]
    src_iu, dst_iu = ei_item_user[0], ei_item_user[1]
    r2 = lambda v: v.reshape(1, C)

    yu, zu = _mm_pre(x_user, W_rel_0_ui.T, W_root_0_iu.T)
    yi, zi = _mm_pre(x_item, W_rel_0_iu.T, W_root_0_ui.T)

    smi, recip_i = _sc_agg_l0(yu, src_ui, dst_ui)
    smu, recip_u = _sc_agg_l0(yi, src_iu, dst_iu)
    yi1, zi1 = _post_mm(smi, zi, r2(b_rel_0_ui),
                        r2(ln_g_0_item), r2(ln_b_0_item),
                        W_rel_1_iu.T, W_root_1_ui.T)
    smu1 = _sc_agg_l1(yi1, src_iu, dst_iu, recip_u)[0]
    yu1, zu1 = _post_mm(smu, zu, r2(b_rel_0_iu),
                        r2(ln_g_0_user), r2(ln_b_0_user),
                        W_rel_1_ui.T, W_root_1_iu.T)
    smi1 = _sc_agg_l1(yu1, src_ui, dst_ui, recip_i)[0]
    out_user = _post(smu1, zu1, r2(b_rel_1_iu),
                     r2(ln_g_1_user), r2(ln_b_1_user))
    out_item = _post(smi1, zi1, r2(b_rel_1_ui),
                     r2(ln_g_1_item), r2(ln_b_1_item))
    return (out_user, out_item)
